# baseline (speedup 1.0000x reference)
import functools

import jax
import jax.numpy as jnp
from jax import lax
from jax.experimental import pallas as pl
from jax.experimental.pallas import tpu as pltpu
from jax.experimental.pallas import tpu_sc as plsc

_NC = 2
_NS = 16
_NW = _NC * _NS
_L = 16


def _sc_pair_dots(src_flat, tgt_flat, src_idx, tgt_idx):
    P = src_idx.shape[0]
    D = src_flat.shape[1]
    ppw = P // _NW
    assert ppw == _L and P % (8 * _NW) == 0

    mesh = plsc.VectorSubcoreMesh(core_axis_name="c", subcore_axis_name="s")
    vec_f32 = jax.ShapeDtypeStruct((P,), jnp.float32)

    @functools.partial(
        pl.kernel,
        out_type=[vec_f32, vec_f32, vec_f32],
        mesh=mesh,
        compiler_params=pltpu.CompilerParams(
            use_tc_tiling_on_sc=False, needs_layout_passes=False),
        scratch_types=[
            pltpu.VMEM((ppw,), jnp.int32),
            pltpu.VMEM((ppw,), jnp.int32),
            pltpu.VMEM((ppw, D), jnp.float32),
            pltpu.VMEM((ppw, D), jnp.float32),
            pltpu.VMEM((ppw,), jnp.float32),
            pltpu.VMEM((ppw,), jnp.float32),
            pltpu.VMEM((ppw,), jnp.float32),
            pltpu.SemaphoreType.DMA,
            pltpu.SemaphoreType.DMA,
        ],
    )
    def sc_kernel(src_hbm, tgt_hbm, sidx_hbm, tidx_hbm,
                  num_hbm, ss_hbm, tt_hbm,
                  idx_s, idx_t, srows, trows, num_v, ss_v, tt_v,
                  sem_s, sem_t):
        wid = lax.axis_index("s") * _NC + lax.axis_index("c")
        base = wid * ppw
        pltpu.sync_copy(sidx_hbm.at[pl.ds(base, ppw)], idx_s)
        pltpu.sync_copy(tidx_hbm.at[pl.ds(base, ppw)], idx_t)
        cs = pltpu.async_copy(src_hbm.at[idx_s], srows, sem_s)
        ct = pltpu.async_copy(tgt_hbm.at[idx_t], trows, sem_t)
        cs.wait()
        ct.wait()

        lanes = lax.iota(jnp.int32, _L)

        def body(d, carry):
            an, ass, att = carry
            dcol = jnp.broadcast_to(d, (_L,))
            vs = plsc.load_gather(srows, [lanes, dcol])
            vt = plsc.load_gather(trows, [lanes, dcol])
            return an + vs * vt, ass + vs * vs, att + vt * vt

        z = jnp.zeros((_L,), jnp.float32)
        an, ass, att = lax.fori_loop(0, D, body, (z, z, z))
        num_v[...] = an
        ss_v[...] = ass
        tt_v[...] = att
        pltpu.sync_copy(num_v, num_hbm.at[pl.ds(base, ppw)])
        pltpu.sync_copy(ss_v, ss_hbm.at[pl.ds(base, ppw)])
        pltpu.sync_copy(tt_v, tt_hbm.at[pl.ds(base, ppw)])

    return sc_kernel(src_flat, tgt_flat, src_idx, tgt_idx)


def _tc_finish_body(num_ref, ss_ref, tt_ref, m_ref, out_ref):
    eps = jnp.float32(1e-8)
    num = num_ref[...]
    den = (jnp.maximum(jnp.sqrt(ss_ref[...]), eps) *
           jnp.maximum(jnp.sqrt(tt_ref[...]), eps))
    m = m_ref[...]
    loss = (1.0 - num / den) * m
    total = jnp.sum(loss)
    n_valid = jnp.sum(m)
    out_ref[0, 0] = jnp.where(
        n_valid > 0, total / jnp.maximum(n_valid, 1.0), jnp.float32(0.0))


def kernel(src_features, tgt_features, src_kps, tgt_kps, valid_mask, patch_size):
    B, H, W, D = src_features.shape
    N = src_kps.shape[1]
    P = B * N

    src_p = (src_kps / patch_size).astype(jnp.int32)
    tgt_p = (tgt_kps / patch_size).astype(jnp.int32)
    sx = jnp.clip(src_p[..., 0], 0, W - 1)
    sy = jnp.clip(src_p[..., 1], 0, H - 1)
    tx = jnp.clip(tgt_p[..., 0], 0, W - 1)
    ty = jnp.clip(tgt_p[..., 1], 0, H - 1)
    b_idx = jnp.arange(B, dtype=jnp.int32)[:, None]
    src_idx = (b_idx * (H * W) + sy * W + sx).reshape(P)
    tgt_idx = (b_idx * (H * W) + ty * W + tx).reshape(P)

    num, ss, tt = _sc_pair_dots(
        src_features.reshape(B * H * W, D),
        tgt_features.reshape(B * H * W, D),
        src_idx, tgt_idx)

    mask_f = valid_mask.reshape(P).astype(jnp.float32)
    rows = P // 128
    out = pl.pallas_call(
        _tc_finish_body,
        out_shape=jax.ShapeDtypeStruct((1, 1), jnp.float32),
        out_specs=pl.BlockSpec(memory_space=pltpu.SMEM),
    )(num.reshape(rows, 128), ss.reshape(rows, 128),
      tt.reshape(rows, 128), mask_f.reshape(rows, 128))
    return out[0, 0]


# speedup vs baseline: 2.4014x; 2.4014x over previous
import functools

import jax
import jax.numpy as jnp
from jax import lax
from jax.experimental import pallas as pl
from jax.experimental.pallas import tpu as pltpu
from jax.experimental.pallas import tpu_sc as plsc

_NC = 2
_NS = 16
_NW = _NC * _NS
_L = 16


def _sc_pair_dots(src_flat, tgt_flat, src_idx, tgt_idx):
    P = src_idx.shape[0]
    D = src_flat.shape[1]
    ppw = P // _NW
    assert ppw == _L and P % (8 * _NW) == 0

    mesh = plsc.VectorSubcoreMesh(core_axis_name="c", subcore_axis_name="s")
    vec_f32 = jax.ShapeDtypeStruct((P,), jnp.float32)

    @functools.partial(
        pl.kernel,
        out_type=[vec_f32, vec_f32, vec_f32],
        mesh=mesh,
        compiler_params=pltpu.CompilerParams(
            use_tc_tiling_on_sc=True, needs_layout_passes=False),
        scratch_types=[
            pltpu.VMEM((ppw,), jnp.int32),
            pltpu.VMEM((ppw,), jnp.int32),
            pltpu.VMEM((ppw, D), jnp.float32),
            pltpu.VMEM((ppw, D), jnp.float32),
            pltpu.VMEM((ppw,), jnp.float32),
            pltpu.VMEM((ppw,), jnp.float32),
            pltpu.VMEM((ppw,), jnp.float32),
            pltpu.SemaphoreType.DMA,
            pltpu.SemaphoreType.DMA,
        ],
    )
    def sc_kernel(src_hbm, tgt_hbm, sidx_hbm, tidx_hbm,
                  num_hbm, ss_hbm, tt_hbm,
                  idx_s, idx_t, srows, trows, num_v, ss_v, tt_v,
                  sem_s, sem_t):
        wid = lax.axis_index("s") * _NC + lax.axis_index("c")
        base = wid * ppw
        pltpu.sync_copy(sidx_hbm.at[pl.ds(base, ppw)], idx_s)
        pltpu.sync_copy(tidx_hbm.at[pl.ds(base, ppw)], idx_t)
        cs = pltpu.async_copy(src_hbm.at[idx_s], srows, sem_s)
        ct = pltpu.async_copy(tgt_hbm.at[idx_t], trows, sem_t)
        cs.wait()
        ct.wait()

        lanes = lax.iota(jnp.int32, _L)

        def body(d, carry):
            an, ass, att = carry
            dcol = jnp.broadcast_to(d, (_L,))
            vs = plsc.load_gather(srows, [lanes, dcol])
            vt = plsc.load_gather(trows, [lanes, dcol])
            return an + vs * vt, ass + vs * vs, att + vt * vt

        z = jnp.zeros((_L,), jnp.float32)
        an, ass, att = lax.fori_loop(0, D, body, (z, z, z))
        num_v[...] = an
        ss_v[...] = ass
        tt_v[...] = att
        pltpu.sync_copy(num_v, num_hbm.at[pl.ds(base, ppw)])
        pltpu.sync_copy(ss_v, ss_hbm.at[pl.ds(base, ppw)])
        pltpu.sync_copy(tt_v, tt_hbm.at[pl.ds(base, ppw)])

    return sc_kernel(src_flat, tgt_flat, src_idx, tgt_idx)


def _tc_finish_body(num_ref, ss_ref, tt_ref, m_ref, out_ref):
    eps = jnp.float32(1e-8)
    num = num_ref[...]
    den = (jnp.maximum(jnp.sqrt(ss_ref[...]), eps) *
           jnp.maximum(jnp.sqrt(tt_ref[...]), eps))
    m = m_ref[...]
    loss = (1.0 - num / den) * m
    total = jnp.sum(loss)
    n_valid = jnp.sum(m)
    out_ref[0, 0] = jnp.where(
        n_valid > 0, total / jnp.maximum(n_valid, 1.0), jnp.float32(0.0))


def kernel(src_features, tgt_features, src_kps, tgt_kps, valid_mask, patch_size):
    B, H, W, D = src_features.shape
    N = src_kps.shape[1]
    P = B * N

    src_p = (src_kps / patch_size).astype(jnp.int32)
    tgt_p = (tgt_kps / patch_size).astype(jnp.int32)
    sx = jnp.clip(src_p[..., 0], 0, W - 1)
    sy = jnp.clip(src_p[..., 1], 0, H - 1)
    tx = jnp.clip(tgt_p[..., 0], 0, W - 1)
    ty = jnp.clip(tgt_p[..., 1], 0, H - 1)
    b_idx = jnp.arange(B, dtype=jnp.int32)[:, None]
    src_idx = (b_idx * (H * W) + sy * W + sx).reshape(P)
    tgt_idx = (b_idx * (H * W) + ty * W + tx).reshape(P)

    num, ss, tt = _sc_pair_dots(
        src_features.reshape(B * H * W, D),
        tgt_features.reshape(B * H * W, D),
        src_idx, tgt_idx)

    mask_f = valid_mask.reshape(P).astype(jnp.float32)
    rows = P // 128
    out = pl.pallas_call(
        _tc_finish_body,
        out_shape=jax.ShapeDtypeStruct((1, 1), jnp.float32),
        out_specs=pl.BlockSpec(memory_space=pltpu.SMEM),
    )(num.reshape(rows, 128), ss.reshape(rows, 128),
      tt.reshape(rows, 128), mask_f.reshape(rows, 128))
    return out[0, 0]


# speedup vs baseline: 2.4248x; 1.0097x over previous
import functools

import jax
import jax.numpy as jnp
from jax import lax
from jax.experimental import pallas as pl
from jax.experimental.pallas import tpu as pltpu
from jax.experimental.pallas import tpu_sc as plsc

_NC = 2
_NS = 16
_NW = _NC * _NS
_L = 16


def _sc_pair_dots(src_flat, tgt_flat, src_idx, tgt_idx):
    P = src_idx.shape[0]
    D = src_flat.shape[1]
    ppw = P // _NW
    assert ppw == _L and P % (8 * _NW) == 0

    mesh = plsc.VectorSubcoreMesh(core_axis_name="c", subcore_axis_name="s")
    vec_f32 = jax.ShapeDtypeStruct((P,), jnp.float32)

    @functools.partial(
        pl.kernel,
        out_type=[vec_f32, vec_f32, vec_f32],
        mesh=mesh,
        compiler_params=pltpu.CompilerParams(
            use_tc_tiling_on_sc=True, needs_layout_passes=False),
        scratch_types=[
            pltpu.VMEM((ppw,), jnp.int32),
            pltpu.VMEM((ppw,), jnp.int32),
            pltpu.VMEM((ppw, D), jnp.float32),
            pltpu.VMEM((ppw, D), jnp.float32),
            pltpu.VMEM((ppw,), jnp.float32),
            pltpu.VMEM((ppw,), jnp.float32),
            pltpu.VMEM((ppw,), jnp.float32),
            pltpu.SemaphoreType.DMA,
            pltpu.SemaphoreType.DMA,
        ],
    )
    def sc_kernel(src_hbm, tgt_hbm, sidx_hbm, tidx_hbm,
                  num_hbm, ss_hbm, tt_hbm,
                  idx_s, idx_t, srows, trows, num_v, ss_v, tt_v,
                  sem_s, sem_t):
        wid = lax.axis_index("s") * _NC + lax.axis_index("c")
        base = wid * ppw
        pltpu.sync_copy(sidx_hbm.at[pl.ds(base, ppw)], idx_s)
        pltpu.sync_copy(tidx_hbm.at[pl.ds(base, ppw)], idx_t)
        cs = pltpu.async_copy(src_hbm.at[idx_s], srows, sem_s)
        ct = pltpu.async_copy(tgt_hbm.at[idx_t], trows, sem_t)
        cs.wait()
        ct.wait()

        lanes = lax.iota(jnp.int32, _L)

        UNROLL = 8
        NACC = 4

        def body(c, carry):
            accs = list(carry)
            d0 = c * UNROLL
            for u in range(UNROLL):
                dcol = jnp.broadcast_to(d0 + u, (_L,))
                vs = plsc.load_gather(srows, [lanes, dcol])
                vt = plsc.load_gather(trows, [lanes, dcol])
                an, ass, att = accs[u % NACC]
                accs[u % NACC] = (an + vs * vt, ass + vs * vs, att + vt * vt)
            return tuple(accs)

        z = jnp.zeros((_L,), jnp.float32)
        accs = lax.fori_loop(0, D // UNROLL, body, tuple((z, z, z) for _ in range(NACC)))
        an = accs[0][0] + accs[1][0] + accs[2][0] + accs[3][0]
        ass = accs[0][1] + accs[1][1] + accs[2][1] + accs[3][1]
        att = accs[0][2] + accs[1][2] + accs[2][2] + accs[3][2]
        num_v[...] = an
        ss_v[...] = ass
        tt_v[...] = att
        pltpu.sync_copy(num_v, num_hbm.at[pl.ds(base, ppw)])
        pltpu.sync_copy(ss_v, ss_hbm.at[pl.ds(base, ppw)])
        pltpu.sync_copy(tt_v, tt_hbm.at[pl.ds(base, ppw)])

    return sc_kernel(src_flat, tgt_flat, src_idx, tgt_idx)


def _tc_finish_body(num_ref, ss_ref, tt_ref, m_ref, out_ref):
    eps = jnp.float32(1e-8)
    num = num_ref[...]
    den = (jnp.maximum(jnp.sqrt(ss_ref[...]), eps) *
           jnp.maximum(jnp.sqrt(tt_ref[...]), eps))
    m = m_ref[...]
    loss = (1.0 - num / den) * m
    total = jnp.sum(loss)
    n_valid = jnp.sum(m)
    out_ref[0, 0] = jnp.where(
        n_valid > 0, total / jnp.maximum(n_valid, 1.0), jnp.float32(0.0))


def kernel(src_features, tgt_features, src_kps, tgt_kps, valid_mask, patch_size):
    B, H, W, D = src_features.shape
    N = src_kps.shape[1]
    P = B * N

    src_p = (src_kps / patch_size).astype(jnp.int32)
    tgt_p = (tgt_kps / patch_size).astype(jnp.int32)
    sx = jnp.clip(src_p[..., 0], 0, W - 1)
    sy = jnp.clip(src_p[..., 1], 0, H - 1)
    tx = jnp.clip(tgt_p[..., 0], 0, W - 1)
    ty = jnp.clip(tgt_p[..., 1], 0, H - 1)
    b_idx = jnp.arange(B, dtype=jnp.int32)[:, None]
    src_idx = (b_idx * (H * W) + sy * W + sx).reshape(P)
    tgt_idx = (b_idx * (H * W) + ty * W + tx).reshape(P)

    num, ss, tt = _sc_pair_dots(
        src_features.reshape(B * H * W, D),
        tgt_features.reshape(B * H * W, D),
        src_idx, tgt_idx)

    mask_f = valid_mask.reshape(P).astype(jnp.float32)
    rows = P // 128
    out = pl.pallas_call(
        _tc_finish_body,
        out_shape=jax.ShapeDtypeStruct((1, 1), jnp.float32),
        out_specs=pl.BlockSpec(memory_space=pltpu.SMEM),
    )(num.reshape(rows, 128), ss.reshape(rows, 128),
      tt.reshape(rows, 128), mask_f.reshape(rows, 128))
    return out[0, 0]


# speedup vs baseline: 3.2684x; 1.3479x over previous
import functools

import jax
import jax.numpy as jnp
from jax import lax
from jax.experimental import pallas as pl
from jax.experimental.pallas import tpu as pltpu
from jax.experimental.pallas import tpu_sc as plsc

_NC = 2
_NS = 16
_NW = _NC * _NS
_L = 16


def _sc_pair_dots(src_flat, tgt_flat, src_idx, tgt_idx):
    P = src_idx.shape[0]
    D = src_flat.shape[1]
    ppw = P // _NW
    assert ppw == _L and P % (8 * _NW) == 0

    mesh = plsc.VectorSubcoreMesh(core_axis_name="c", subcore_axis_name="s")
    vec_f32 = jax.ShapeDtypeStruct((P,), jnp.float32)

    @functools.partial(
        pl.kernel,
        out_type=[vec_f32, vec_f32, vec_f32],
        mesh=mesh,
        compiler_params=pltpu.CompilerParams(
            use_tc_tiling_on_sc=True, needs_layout_passes=False),
        scratch_types=[
            pltpu.VMEM((ppw,), jnp.int32),
            pltpu.VMEM((ppw,), jnp.int32),
            pltpu.VMEM((ppw, D), jnp.float32),
            pltpu.VMEM((ppw, D), jnp.float32),
            pltpu.VMEM((ppw,), jnp.float32),
            pltpu.VMEM((ppw,), jnp.float32),
            pltpu.VMEM((ppw,), jnp.float32),
            pltpu.SemaphoreType.DMA,
            pltpu.SemaphoreType.DMA,
        ],
    )
    def sc_kernel(src_hbm, tgt_hbm, sidx_hbm, tidx_hbm,
                  num_hbm, ss_hbm, tt_hbm,
                  idx_s, idx_t, srows, trows, num_v, ss_v, tt_v,
                  sem_s, sem_t):
        wid = lax.axis_index("s") * _NC + lax.axis_index("c")
        base = wid * ppw
        pltpu.sync_copy(sidx_hbm.at[pl.ds(base, ppw)], idx_s)
        pltpu.sync_copy(tidx_hbm.at[pl.ds(base, ppw)], idx_t)
        cs = pltpu.async_copy(src_hbm.at[idx_s], srows, sem_s)
        ct = pltpu.async_copy(tgt_hbm.at[idx_t], trows, sem_t)
        cs.wait()
        ct.wait()

        lanes = lax.iota(jnp.int32, _L)

        UNROLL = 8
        NACC = 4

        def body(c, carry):
            accs = list(carry[0])
            dcol = carry[1]
            for u in range(UNROLL):
                vs = plsc.load_gather(srows, [lanes, dcol])
                vt = plsc.load_gather(trows, [lanes, dcol])
                an, ass, att = accs[u % NACC]
                accs[u % NACC] = (an + vs * vt, ass + vs * vs, att + vt * vt)
                dcol = dcol + 1
                dcol = jnp.where(dcol >= D, dcol - D, dcol)
            return tuple(accs), dcol

        z = jnp.zeros((_L,), jnp.float32)
        accs, _ = lax.fori_loop(
            0, D // UNROLL, body,
            (tuple((z, z, z) for _ in range(NACC)), lanes))
        an = accs[0][0] + accs[1][0] + accs[2][0] + accs[3][0]
        ass = accs[0][1] + accs[1][1] + accs[2][1] + accs[3][1]
        att = accs[0][2] + accs[1][2] + accs[2][2] + accs[3][2]
        num_v[...] = an
        ss_v[...] = ass
        tt_v[...] = att
        pltpu.sync_copy(num_v, num_hbm.at[pl.ds(base, ppw)])
        pltpu.sync_copy(ss_v, ss_hbm.at[pl.ds(base, ppw)])
        pltpu.sync_copy(tt_v, tt_hbm.at[pl.ds(base, ppw)])

    return sc_kernel(src_flat, tgt_flat, src_idx, tgt_idx)


def _tc_finish_body(num_ref, ss_ref, tt_ref, m_ref, out_ref):
    eps = jnp.float32(1e-8)
    num = num_ref[...]
    den = (jnp.maximum(jnp.sqrt(ss_ref[...]), eps) *
           jnp.maximum(jnp.sqrt(tt_ref[...]), eps))
    m = m_ref[...]
    loss = (1.0 - num / den) * m
    total = jnp.sum(loss)
    n_valid = jnp.sum(m)
    out_ref[0, 0] = jnp.where(
        n_valid > 0, total / jnp.maximum(n_valid, 1.0), jnp.float32(0.0))


def kernel(src_features, tgt_features, src_kps, tgt_kps, valid_mask, patch_size):
    B, H, W, D = src_features.shape
    N = src_kps.shape[1]
    P = B * N

    src_p = (src_kps / patch_size).astype(jnp.int32)
    tgt_p = (tgt_kps / patch_size).astype(jnp.int32)
    sx = jnp.clip(src_p[..., 0], 0, W - 1)
    sy = jnp.clip(src_p[..., 1], 0, H - 1)
    tx = jnp.clip(tgt_p[..., 0], 0, W - 1)
    ty = jnp.clip(tgt_p[..., 1], 0, H - 1)
    b_idx = jnp.arange(B, dtype=jnp.int32)[:, None]
    src_idx = (b_idx * (H * W) + sy * W + sx).reshape(P)
    tgt_idx = (b_idx * (H * W) + ty * W + tx).reshape(P)

    num, ss, tt = _sc_pair_dots(
        src_features.reshape(B * H * W, D),
        tgt_features.reshape(B * H * W, D),
        src_idx, tgt_idx)

    mask_f = valid_mask.reshape(P).astype(jnp.float32)
    rows = P // 128
    out = pl.pallas_call(
        _tc_finish_body,
        out_shape=jax.ShapeDtypeStruct((1, 1), jnp.float32),
        out_specs=pl.BlockSpec(memory_space=pltpu.SMEM),
    )(num.reshape(rows, 128), ss.reshape(rows, 128),
      tt.reshape(rows, 128), mask_f.reshape(rows, 128))
    return out[0, 0]
